# Initial kernel scaffold; baseline (speedup 1.0000x reference)
import jax
import jax.numpy as jnp
from jax.experimental import pallas as pl


def kernel(x, token_to_slot, lora_a, lora_b, lora_scaling):
    raise NotImplementedError("write your pallas kernel here")


# speedup vs baseline: 8.7577x; 8.7577x over previous
import functools

import jax
import jax.numpy as jnp
from jax.experimental import pallas as pl


def _lora_block_kernel(slot_ref, x_ref, a_ref, b_ref, scal_ref, out_ref, *, rank):
    h = jnp.dot(x_ref[...], a_ref[...], preferred_element_type=jnp.float32)
    er = h.shape[-1]
    col_slot = jax.lax.broadcasted_iota(jnp.int32, (1, er), 1) // rank
    maskf = jnp.where(slot_ref[...] == col_slot, scal_ref[...], 0.0)
    out_ref[...] = jnp.dot(h * maskf, b_ref[...], preferred_element_type=jnp.float32)


def kernel(x, token_to_slot, lora_a, lora_b, lora_scaling):
    T, D = x.shape
    E, _, R = lora_a.shape
    Dout = lora_b.shape[-1]
    ER = E * R

    a_cat = jnp.transpose(lora_a, (1, 0, 2)).reshape(D, ER)
    b_cat = lora_b.reshape(ER, Dout)
    scal = jnp.repeat(lora_scaling, R).reshape(1, ER)
    slot2d = token_to_slot.reshape(T, 1)

    BT = 512
    grid = (T // BT,)
    return pl.pallas_call(
        functools.partial(_lora_block_kernel, rank=R),
        grid=grid,
        in_specs=[
            pl.BlockSpec((BT, 1), lambda i: (i, 0)),
            pl.BlockSpec((BT, D), lambda i: (i, 0)),
            pl.BlockSpec((D, ER), lambda i: (0, 0)),
            pl.BlockSpec((ER, Dout), lambda i: (0, 0)),
            pl.BlockSpec((1, ER), lambda i: (0, 0)),
        ],
        out_specs=pl.BlockSpec((BT, Dout), lambda i: (i, 0)),
        out_shape=jax.ShapeDtypeStruct((T, Dout), x.dtype),
    )(slot2d, x, a_cat, b_cat, scal)
